# Initial kernel scaffold; baseline (speedup 1.0000x reference)
import jax
import jax.numpy as jnp
from jax.experimental import pallas as pl


def kernel(x, edge_index, edge_attr, temporal_features, lagged_targets, params):
    raise NotImplementedError("write your pallas kernel here")


# speedup vs baseline: 1.1629x; 1.1629x over previous
import functools

import jax
import jax.numpy as jnp
from jax.experimental import pallas as pl
from jax.experimental.pallas import tpu as pltpu

N_LAG_E = 3
H = 64


def _ln(x, g, b):
    m = jnp.mean(x, axis=-1, keepdims=True)
    v = jnp.var(x, axis=-1, keepdims=True)
    return (x - m) / jnp.sqrt(v + 1e-5) * g + b


def _gru(seq, Wi, Wh, bi, bh):
    n, L, hh = seq.shape
    h = jnp.zeros((n, hh), dtype=seq.dtype)
    outs = []
    for t in range(L):
        gi = seq[:, t, :] @ Wi + bi
        gh = h @ Wh + bh
        ir, iz, inn = jnp.split(gi, 3, axis=-1)
        hr, hz, hn = jnp.split(gh, 3, axis=-1)
        r = jax.nn.sigmoid(ir + hr)
        z = jax.nn.sigmoid(iz + hz)
        ng = jnp.tanh(inn + r * hn)
        h = (1.0 - z) * ng + z * h
        outs.append(h)
    return jnp.stack(outs, axis=1)


def _temporal_mem(cur, bank, p):
    if bank is None:
        return cur, jnp.zeros((cur.shape[0], N_LAG_E, cur.shape[1]), cur.dtype)
    out0 = _gru(bank, p['gru_Wi0'], p['gru_Wh0'], p['gru_bi0'], p['gru_bh0'])
    out1 = _gru(out0, p['gru_Wi1'], p['gru_Wh1'], p['gru_bi1'], p['gru_bh1'])
    exp_cur = jnp.broadcast_to(cur[:, None, :], out1.shape)
    ai = jnp.concatenate([exp_cur, out1], axis=-1)
    hid = jnp.tanh(ai @ p['ma_W1'] + p['ma_b1'])
    logits = hid @ p['ma_W2'] + p['ma_b2']
    w = jax.nn.softmax(logits, axis=1)
    weighted = jnp.sum(out1 * w, axis=1)
    new_bank = jnp.concatenate([bank[:, 1:, :], cur[:, None, :]], axis=1)
    return cur + weighted, new_bank


def _gat(x, src, dst, W, a_src, a_dst, bias, heads, oc):
    n = x.shape[0]
    h = (x @ W).reshape(n, heads, oc)
    es = jnp.sum(h * a_src[None, :, :], axis=-1)
    ed = jnp.sum(h * a_dst[None, :, :], axis=-1)
    e = jax.nn.leaky_relu(es[src] + ed[dst], 0.2)
    p = jnp.exp(e)
    s = jax.ops.segment_sum(p, dst, num_segments=n)
    out = jax.ops.segment_sum(h[src] * p[:, :, None], dst, num_segments=n)
    out = out / (s[:, :, None] + 1e-16)
    return out.reshape(n, heads * oc) + bias


def _tail_body(xt_ref, c_ref, fuW_ref, fg_ref, fb_ref, o1W_ref, o1b_ref,
               o1g_ref, o1bb_ref, o2W_ref, o2b_ref, out_ref):
    T = xt_ref.shape[0]
    for t in range(T):
        xb = xt_ref[t]
        y = jnp.dot(xb, fuW_ref[...], preferred_element_type=jnp.float32) + c_ref[t]
        m = jnp.mean(y, axis=-1, keepdims=True)
        v = jnp.mean((y - m) ** 2, axis=-1, keepdims=True)
        y = (y - m) * jax.lax.rsqrt(v + 1e-5) * fg_ref[...] + fb_ref[...]
        y = jnp.where(y > 0, y, jnp.exp(y) - 1.0)
        z = jnp.dot(y, o1W_ref[...], preferred_element_type=jnp.float32) + o1b_ref[...]
        m = jnp.mean(z, axis=-1, keepdims=True)
        v = jnp.mean((z - m) ** 2, axis=-1, keepdims=True)
        z = (z - m) * jax.lax.rsqrt(v + 1e-5) * o1g_ref[...] + o1bb_ref[...]
        z = jnp.where(z > 0, z, jnp.exp(z) - 1.0)
        o = jnp.dot(z, o2W_ref[...], preferred_element_type=jnp.float32)
        out_ref[0, t] = o[:, 0] + o2b_ref[0]


def _tail(xt_all, consts, p):
    T, n, _ = xt_all.shape
    BLK = 2000
    NB = n // BLK
    out = pl.pallas_call(
        _tail_body,
        grid=(NB,),
        in_specs=[
            pl.BlockSpec((T, BLK, H), lambda i: (0, i, 0)),
            pl.BlockSpec((T, H), lambda i: (0, 0)),
            pl.BlockSpec((H, H), lambda i: (0, 0)),
            pl.BlockSpec((H,), lambda i: (0,)),
            pl.BlockSpec((H,), lambda i: (0,)),
            pl.BlockSpec((H, H // 2), lambda i: (0, 0)),
            pl.BlockSpec((H // 2,), lambda i: (0,)),
            pl.BlockSpec((H // 2,), lambda i: (0,)),
            pl.BlockSpec((H // 2,), lambda i: (0,)),
            pl.BlockSpec((H // 2, 1), lambda i: (0, 0)),
            pl.BlockSpec((1,), lambda i: (0,)),
        ],
        out_specs=pl.BlockSpec((1, T, BLK), lambda i: (i, 0, 0)),
        out_shape=jax.ShapeDtypeStruct((NB, T, BLK), jnp.float32),
    )(xt_all, consts, p['fu_W'][:H], p['fu_ln_g'], p['fu_ln_b'],
      p['o1_W'], p['o1_b'], p['o1_ln_g'], p['o1_ln_b'], p['o2_W'], p['o2_b'])
    return out.transpose(1, 0, 2).reshape(T, n)


def kernel(x, edge_index, edge_attr, temporal_features, lagged_targets, params):
    p = params
    src = edge_index[0]
    dst = edge_index[1]
    n = x.shape[0]
    T = temporal_features.shape[0]

    tf = temporal_features
    tp0 = jax.nn.elu(_ln(tf @ p['tn_W'] + p['tn_b'], p['tn_ln_g'], p['tn_ln_b']))
    bank = None
    tps = []
    for t in range(T):
        cur, bank = _temporal_mem(tp0[t:t + 1], bank, p)
        tps.append(cur[0])
    tp_all = jnp.stack(tps)

    xW = x @ p['emb_W'][:x.shape[1]]
    W_tf = p['emb_W'][x.shape[1]:x.shape[1] + tf.shape[1]]
    W_lag = p['emb_W'][x.shape[1] + tf.shape[1]:]
    tf_part = tf @ W_tf + p['emb_b']

    xts = []
    for t in range(T):
        xt = xW + lagged_targets[t] @ W_lag + tf_part[t]
        xt = jax.nn.elu(_ln(xt, p['emb_ln_g'], p['emb_ln_b']))
        xt = jax.nn.relu(_gat(xt, src, dst, p['g1_W'], p['g1_as'], p['g1_ad'], p['g1_b'], 4, H))
        xt = jax.nn.relu(_gat(xt, src, dst, p['g2_W'], p['g2_as'], p['g2_ad'], p['g2_b'], 1, H))
        xts.append(xt)
    xt_all = jnp.stack(xts)

    consts = tp_all @ p['fu_W'][H:] + p['fu_b']
    return _tail(xt_all, consts, p)


# speedup vs baseline: 17.6935x; 15.2152x over previous
import functools

import jax
import jax.numpy as jnp
from jax import lax
from jax.experimental import pallas as pl
from jax.experimental.pallas import tpu as pltpu
from jax.experimental.pallas import tpu_sc as plsc

N_LAG_E = 3
H = 64
T = 4
BLK = 1000

C = 256
CD = C + 8
NCH = 196
NPAD = NCH * C

D1 = 1024
D2 = 256
KE = 64


def _ln(x, g, b):
    m = jnp.mean(x, axis=-1, keepdims=True)
    v = jnp.var(x, axis=-1, keepdims=True)
    return (x - m) / jnp.sqrt(v + 1e-5) * g + b


def _gru(seq, Wi, Wh, bi, bh):
    n, L, hh = seq.shape
    h = jnp.zeros((n, hh), dtype=seq.dtype)
    outs = []
    for t in range(L):
        gi = seq[:, t, :] @ Wi + bi
        gh = h @ Wh + bh
        ir, iz, inn = jnp.split(gi, 3, axis=-1)
        hr, hz, hn = jnp.split(gh, 3, axis=-1)
        r = jax.nn.sigmoid(ir + hr)
        z = jax.nn.sigmoid(iz + hz)
        ng = jnp.tanh(inn + r * hn)
        h = (1.0 - z) * ng + z * h
        outs.append(h)
    return jnp.stack(outs, axis=1)


def _temporal_mem(cur, bank, p):
    if bank is None:
        return cur, jnp.zeros((cur.shape[0], N_LAG_E, cur.shape[1]), cur.dtype)
    out0 = _gru(bank, p['gru_Wi0'], p['gru_Wh0'], p['gru_bi0'], p['gru_bh0'])
    out1 = _gru(out0, p['gru_Wi1'], p['gru_Wh1'], p['gru_bi1'], p['gru_bh1'])
    exp_cur = jnp.broadcast_to(cur[:, None, :], out1.shape)
    ai = jnp.concatenate([exp_cur, out1], axis=-1)
    hid = jnp.tanh(ai @ p['ma_W1'] + p['ma_b1'])
    logits = hid @ p['ma_W2'] + p['ma_b2']
    w = jax.nn.softmax(logits, axis=1)
    weighted = jnp.sum(out1 * w, axis=1)
    new_bank = jnp.concatenate([bank[:, 1:, :], cur[:, None, :]], axis=1)
    return cur + weighted, new_bank


def _elu(x):
    return jnp.where(x > 0, x, jnp.exp(x) - 1.0)


def _roll_cols(x, k):
    k = k % 16
    if k == 0:
        return x
    return jnp.concatenate([x[:, k:], x[:, :k]], axis=1)


def _pre_body(x_ref, lag_ref, tfp_ref, lng_ref, lnb_ref, embWx_ref, Wlag_ref,
              g1W_ref, A1_ref, t1_ref, ed1_ref):
    xw = jnp.dot(x_ref[...], embWx_ref[...], preferred_element_type=jnp.float32)
    hs, es_c, ed_c = [], [], []
    for t in range(T):
        xt = xw + jnp.dot(lag_ref[t], Wlag_ref[...],
                          preferred_element_type=jnp.float32) + tfp_ref[t]
        m = jnp.mean(xt, axis=-1, keepdims=True)
        v = jnp.mean((xt - m) ** 2, axis=-1, keepdims=True)
        xt = (xt - m) * lax.rsqrt(v + 1e-5) * lng_ref[...] + lnb_ref[...]
        xt = _elu(xt)
        h = jnp.dot(xt, g1W_ref[...], preferred_element_type=jnp.float32)
        e8 = jnp.dot(h, A1_ref[...], preferred_element_type=jnp.float32)
        hs.append(h)
        es_c.append(e8[:, :4])
        ed_c.append(e8[:, 4:])
    es16 = jnp.concatenate(es_c, 1)
    es_rot = jnp.concatenate([_roll_cols(es16, 2 * r) for r in range(8)], 1)
    t1_ref[...] = jnp.concatenate(hs + [es_rot], 1)
    ed1_ref[...] = jnp.concatenate(ed_c, 1)


def _pre(x, lagged, tf_part, p, A1):
    n = x.shape[0]
    nb = n // BLK
    return pl.pallas_call(
        _pre_body,
        grid=(nb,),
        in_specs=[
            pl.BlockSpec((BLK, 128), lambda i: (i, 0)),
            pl.BlockSpec((T, BLK, 3), lambda i: (0, i, 0)),
            pl.BlockSpec((T, H), lambda i: (0, 0)),
            pl.BlockSpec((H,), lambda i: (0,)),
            pl.BlockSpec((H,), lambda i: (0,)),
            pl.BlockSpec((128, H), lambda i: (0, 0)),
            pl.BlockSpec((3, H), lambda i: (0, 0)),
            pl.BlockSpec((H, 4 * H), lambda i: (0, 0)),
            pl.BlockSpec((4 * H, 8), lambda i: (0, 0)),
        ],
        out_specs=[
            pl.BlockSpec((BLK, D1 + 128), lambda i: (i, 0)),
            pl.BlockSpec((BLK, 16), lambda i: (i, 0)),
        ],
        out_shape=[
            jax.ShapeDtypeStruct((n, D1 + 128), jnp.float32),
            jax.ShapeDtypeStruct((n, 16), jnp.float32),
        ],
    )(x, lagged, tf_part, p['emb_ln_g'], p['emb_ln_b'],
      p['emb_W'][:128], p['emb_W'][144:], p['g1_W'], A1)


def _mid_body(agg_ref, s_ref, b1_ref, g2W_ref, A2_ref, t2_ref, ed2_ref):
    blk = agg_ref.shape[0]
    sall = s_ref[...]
    h2s, es_c, ed_c = [], [], []
    for t in range(T):
        a = agg_ref[:, t * 256:(t + 1) * 256]
        s4 = jnp.concatenate(
            [sall[(t * 4 + h) // 2, :, (t * 4 + h) % 2:(t * 4 + h) % 2 + 1]
             for h in range(4)], 1)
        sb = jnp.broadcast_to(s4[:, :, None], (blk, 4, 64)).reshape(blk, 256)
        xt1 = jnp.maximum(a / (sb + 1e-16) + b1_ref[...], 0.0)
        h2 = jnp.dot(xt1, g2W_ref[...], preferred_element_type=jnp.float32)
        e2 = jnp.dot(h2, A2_ref[...], preferred_element_type=jnp.float32)
        h2s.append(h2)
        es_c.append(e2[:, :1])
        ed_c.append(e2[:, 1:])
    z12 = jnp.zeros((blk, 12), jnp.float32)
    es16 = jnp.concatenate(es_c + [z12], 1)
    es_rot = jnp.concatenate(
        [_roll_cols(es16, 2 * (r % 2)) for r in range(8)], 1)
    t2_ref[...] = jnp.concatenate(h2s + [es_rot], 1)
    ed2_ref[...] = jnp.concatenate(ed_c + [z12], 1)


def _mid(agg1, s1, p, A2, n):
    nb = n // BLK
    return pl.pallas_call(
        _mid_body,
        grid=(nb,),
        in_specs=[
            pl.BlockSpec((BLK, D1), lambda i: (i, 0)),
            pl.BlockSpec((8, BLK, 16), lambda i: (0, i, 0)),
            pl.BlockSpec((4 * H,), lambda i: (0,)),
            pl.BlockSpec((4 * H, H), lambda i: (0, 0)),
            pl.BlockSpec((H, 2), lambda i: (0, 0)),
        ],
        out_specs=[
            pl.BlockSpec((BLK, D2 + 128), lambda i: (i, 0)),
            pl.BlockSpec((BLK, 16), lambda i: (i, 0)),
        ],
        out_shape=[
            jax.ShapeDtypeStruct((n, D2 + 128), jnp.float32),
            jax.ShapeDtypeStruct((n, 16), jnp.float32),
        ],
    )(agg1, s1, p['g1_b'], p['g2_W'], A2)


def _tail_body(agg_ref, s_ref, g2b_ref, c_ref, fuW_ref, fg_ref, fb_ref,
               o1W_ref, o1b_ref, o1g_ref, o1bb_ref, o2W_ref, o2b_ref, out_ref):
    sall = s_ref[...]
    for t in range(T):
        a = agg_ref[:, t * 64:(t + 1) * 64]
        s = sall[t // 2, :, t % 2:t % 2 + 1]
        xt2 = jnp.maximum(a / (s + 1e-16) + g2b_ref[...], 0.0)
        y = jnp.dot(xt2, fuW_ref[...], preferred_element_type=jnp.float32) + c_ref[t]
        m = jnp.mean(y, axis=-1, keepdims=True)
        v = jnp.mean((y - m) ** 2, axis=-1, keepdims=True)
        y = (y - m) * lax.rsqrt(v + 1e-5) * fg_ref[...] + fb_ref[...]
        y = _elu(y)
        z = jnp.dot(y, o1W_ref[...], preferred_element_type=jnp.float32) + o1b_ref[...]
        m = jnp.mean(z, axis=-1, keepdims=True)
        v = jnp.mean((z - m) ** 2, axis=-1, keepdims=True)
        z = (z - m) * lax.rsqrt(v + 1e-5) * o1g_ref[...] + o1bb_ref[...]
        z = _elu(z)
        o = jnp.dot(z, o2W_ref[...], preferred_element_type=jnp.float32)
        out_ref[0, t] = o[:, 0] + o2b_ref[0]


def _tail(agg2, s2, consts, p, n):
    nb = n // BLK
    out = pl.pallas_call(
        _tail_body,
        grid=(nb,),
        in_specs=[
            pl.BlockSpec((BLK, D2), lambda i: (i, 0)),
            pl.BlockSpec((2, BLK, 16), lambda i: (0, i, 0)),
            pl.BlockSpec((H,), lambda i: (0,)),
            pl.BlockSpec((T, H), lambda i: (0, 0)),
            pl.BlockSpec((H, H), lambda i: (0, 0)),
            pl.BlockSpec((H,), lambda i: (0,)),
            pl.BlockSpec((H,), lambda i: (0,)),
            pl.BlockSpec((H, H // 2), lambda i: (0, 0)),
            pl.BlockSpec((H // 2,), lambda i: (0,)),
            pl.BlockSpec((H // 2,), lambda i: (0,)),
            pl.BlockSpec((H // 2,), lambda i: (0,)),
            pl.BlockSpec((H // 2, 1), lambda i: (0, 0)),
            pl.BlockSpec((1,), lambda i: (0,)),
        ],
        out_specs=pl.BlockSpec((1, T, BLK), lambda i: (i, 0, 0)),
        out_shape=jax.ShapeDtypeStruct((nb, T, BLK), jnp.float32),
    )(agg2, s2, p['g2_b'], consts, p['fu_W'][:H], p['fu_ln_g'], p['fu_ln_b'],
      p['o1_W'], p['o1_b'], p['o1_ln_g'], p['o1_ln_b'], p['o2_W'], p['o2_b'])
    return out.transpose(1, 0, 2).reshape(T, n)


def _make_gat_agg(NCB):
    RPN = NCB + 1
    NG = 2 * (16 // NCB)
    mesh = plsc.VectorSubcoreMesh(core_axis_name="c", subcore_axis_name="s")

    @functools.partial(
        pl.kernel,
        out_type=(jax.ShapeDtypeStruct((NPAD, NCB, 128), jnp.float32),
                  jax.ShapeDtypeStruct((NCB, NPAD, 16), jnp.float32)),
        mesh=mesh,
        scratch_types=[
            pltpu.VMEM((KE,), jnp.int32),
            pltpu.VMEM((KE,), jnp.int32),
            pltpu.VMEM((KE,), jnp.int32),
            pltpu.VMEM((KE,), jnp.int32),
            pltpu.VMEM((KE, 128), jnp.float32),
            pltpu.VMEM((KE, 128), jnp.float32),
            pltpu.VMEM((C, 16), jnp.float32),
            pltpu.VMEM((224,), jnp.int32),
            pltpu.VMEM((CD, 128), jnp.float32),
            pltpu.VMEM((CD, 16), jnp.float32),
            pltpu.SemaphoreType.DMA,
        ],
    )
    def k(table, ed_t, srcs, dsts, offs, out, out_s,
          src_v, dst_v, idx_v, idxe_v, rows_v, es_v, ed_v, offs_v,
          acc, s_acc, sem):
        cid = lax.axis_index("c")
        sid = lax.axis_index("s")
        cb = sid % NCB
        eh = sid // NCB
        gidx = cid * (16 // NCB) + eh
        lanes = jnp.arange(16, dtype=jnp.int32)
        z16 = jnp.zeros((16,), jnp.float32)
        d01 = jnp.where(lanes < 2, 1.0, 0.0)
        pltpu.sync_copy(offs.at[pl.ds(0, 224)], offs_v)
        nch_mine = (NCH + NG - 1 - gidx) // NG

        def chunk(i, carry):
            ch = i * NG + gidx
            cbase = ch * C

            def zr(r, c):
                for q in range(8):
                    acc[r, pl.ds(q * 16, 16)] = z16
                s_acc[r] = z16
                return c

            lax.fori_loop(0, C, zr, 0)
            pltpu.sync_copy(ed_t.at[sid, pl.ds(cbase, C)], ed_v)
            ovec = offs_v[pl.ds(ch, 16)]
            o0 = ovec[0]
            o1 = ovec[1]
            w0 = (o0 // 8) * 8
            nb = (o1 - w0 + KE - 1) // KE

            def batch(b, c2):
                base = pl.multiple_of(w0 + b * KE, 8)
                pltpu.sync_copy(srcs.at[pl.ds(base, KE)], src_v)
                pltpu.sync_copy(dsts.at[pl.ds(base, KE)], dst_v)
                for g in range(KE // 16):
                    sv = src_v[pl.ds(g * 16, 16)]
                    idx_v[pl.ds(g * 16, 16)] = sv * RPN + cb
                    idxe_v[pl.ds(g * 16, 16)] = sv * RPN + NCB
                cp = pltpu.make_async_copy(table.at[idx_v], rows_v, sem)
                cp.start()
                cp.wait()
                cp2 = pltpu.make_async_copy(table.at[idxe_v], es_v, sem)
                cp2.start()
                cp2.wait()

                def group(g, c3):
                    dv = dst_v[pl.ds(g * 16, 16)] - cbase
                    pos = base + g * 16 + lanes
                    m = (dv >= 0) & (dv < C) & (pos < o1)
                    ld16 = jnp.where(m, dv, C)
                    ldc = jnp.minimum(ld16, C - 1)
                    for l in range(16):
                        kk = g * 16 + l
                        r = ld16[l]
                        rc = ldc[l]
                        ev = es_v[kk, pl.ds(0, 16)] + ed_v[rc]
                        pv = jnp.exp(jnp.maximum(ev, 0.2 * ev))
                        s0 = pv[0]
                        s1 = pv[1]
                        for q in range(4):
                            sl = pl.ds(q * 16, 16)
                            plsc.addupdate(acc.at[r, sl], rows_v[kk, sl] * s0)
                        for q in range(4, 8):
                            sl = pl.ds(q * 16, 16)
                            plsc.addupdate(acc.at[r, sl], rows_v[kk, sl] * s1)
                        plsc.addupdate(s_acc.at[r], pv * d01)
                    return c3

                lax.fori_loop(0, KE // 16, group, 0)
                return c2

            lax.fori_loop(0, nb, batch, 0)
            pltpu.sync_copy(acc.at[pl.ds(0, C)], out.at[pl.ds(cbase, C), cb])
            pltpu.sync_copy(s_acc.at[pl.ds(0, C)], out_s.at[cb, pl.ds(cbase, C)])
            return carry

        lax.fori_loop(0, nch_mine, chunk, 0)

    return k


def kernel(x, edge_index, edge_attr, temporal_features, lagged_targets, params):
    p = params
    src = edge_index[0]
    dst = edge_index[1]
    n = x.shape[0]

    tf = temporal_features
    tp0 = jax.nn.elu(_ln(tf @ p['tn_W'] + p['tn_b'], p['tn_ln_g'], p['tn_ln_b']))
    bank = None
    tps = []
    for t in range(T):
        cur, bank = _temporal_mem(tp0[t:t + 1], bank, p)
        tps.append(cur[0])
    tp_all = jnp.stack(tps)
    consts = tp_all @ p['fu_W'][H:] + p['fu_b']

    tf_part = tf @ p['emb_W'][128:144] + p['emb_b']
    idx = jnp.arange(256)
    hd_idx = idx // 64
    ch_idx = idx % 64
    A1 = jnp.zeros((256, 8), jnp.float32)
    A1 = A1.at[idx, hd_idx].set(p['g1_as'][hd_idx, ch_idx])
    A1 = A1.at[idx, hd_idx + 4].set(p['g1_ad'][hd_idx, ch_idx])
    A2 = jnp.concatenate([p['g2_as'].T, p['g2_ad'].T], axis=1)

    dst_s, perm = lax.sort_key_val(dst, jnp.arange(dst.shape[0], dtype=jnp.int32))
    src_s = src[perm]
    pad = jnp.zeros((128,), jnp.int32)
    srcs_p = jnp.concatenate([src_s, pad])
    dsts_p = jnp.concatenate([dst_s, pad])
    bounds = jnp.arange(NCH + 1, dtype=jnp.int32) * C
    offs = jnp.searchsorted(dst_s, bounds).astype(jnp.int32)
    offs = jnp.concatenate([offs, jnp.full((224 - NCH - 1,), dst.shape[0], jnp.int32)])

    table1, ed1 = _pre(x, lagged_targets, tf_part, p, A1)
    ed1p = jnp.zeros((NPAD, 16), jnp.float32).at[:n].set(ed1)
    ed1r = jnp.stack([jnp.roll(ed1p, -2 * (s % 8), axis=1) for s in range(16)],
                     axis=0)
    agg1, s1 = _make_gat_agg(8)(
        table1.reshape(n * 9, 128), ed1r, srcs_p, dsts_p, offs)
    table2, ed2 = _mid(agg1.reshape(NPAD, D1)[:n], s1[:, :n], p, A2, n)
    ed2p = jnp.zeros((NPAD, 16), jnp.float32).at[:n].set(ed2)
    ed2r = jnp.stack([jnp.roll(ed2p, -2 * (s % 2), axis=1) for s in range(16)],
                     axis=0)
    agg2, s2 = _make_gat_agg(2)(
        table2.reshape(n * 3, 128), ed2r, srcs_p, dsts_p, offs)
    return _tail(agg2.reshape(NPAD, D2)[:n], s2[:, :n], consts, p, n)


# speedup vs baseline: 19.8289x; 1.1207x over previous
import functools

import jax
import jax.numpy as jnp
from jax import lax
from jax.experimental import pallas as pl
from jax.experimental.pallas import tpu as pltpu
from jax.experimental.pallas import tpu_sc as plsc

N_LAG_E = 3
H = 64
T = 4
BLK = 1000

C = 256
CD = C + 8
NCH = 196
NPAD = NCH * C

D1 = 1024
D2 = 256
KE = 64


def _ln(x, g, b):
    m = jnp.mean(x, axis=-1, keepdims=True)
    v = jnp.var(x, axis=-1, keepdims=True)
    return (x - m) / jnp.sqrt(v + 1e-5) * g + b


def _gru(seq, Wi, Wh, bi, bh):
    n, L, hh = seq.shape
    h = jnp.zeros((n, hh), dtype=seq.dtype)
    outs = []
    for t in range(L):
        gi = seq[:, t, :] @ Wi + bi
        gh = h @ Wh + bh
        ir, iz, inn = jnp.split(gi, 3, axis=-1)
        hr, hz, hn = jnp.split(gh, 3, axis=-1)
        r = jax.nn.sigmoid(ir + hr)
        z = jax.nn.sigmoid(iz + hz)
        ng = jnp.tanh(inn + r * hn)
        h = (1.0 - z) * ng + z * h
        outs.append(h)
    return jnp.stack(outs, axis=1)


def _temporal_mem(cur, bank, p):
    if bank is None:
        return cur, jnp.zeros((cur.shape[0], N_LAG_E, cur.shape[1]), cur.dtype)
    out0 = _gru(bank, p['gru_Wi0'], p['gru_Wh0'], p['gru_bi0'], p['gru_bh0'])
    out1 = _gru(out0, p['gru_Wi1'], p['gru_Wh1'], p['gru_bi1'], p['gru_bh1'])
    exp_cur = jnp.broadcast_to(cur[:, None, :], out1.shape)
    ai = jnp.concatenate([exp_cur, out1], axis=-1)
    hid = jnp.tanh(ai @ p['ma_W1'] + p['ma_b1'])
    logits = hid @ p['ma_W2'] + p['ma_b2']
    w = jax.nn.softmax(logits, axis=1)
    weighted = jnp.sum(out1 * w, axis=1)
    new_bank = jnp.concatenate([bank[:, 1:, :], cur[:, None, :]], axis=1)
    return cur + weighted, new_bank


def _elu(x):
    return jnp.where(x > 0, x, jnp.exp(x) - 1.0)


def _roll_cols(x, k):
    k = k % 16
    if k == 0:
        return x
    return jnp.concatenate([x[:, k:], x[:, :k]], axis=1)


def _pre_body(x_ref, lag_ref, tfp_ref, lng_ref, lnb_ref, embWx_ref, Wlag_ref,
              g1W_ref, A1_ref, t1_ref, ed1_ref):
    xw = jnp.dot(x_ref[...], embWx_ref[...], preferred_element_type=jnp.float32)
    hs, es_c, ed_c = [], [], []
    for t in range(T):
        xt = xw + jnp.dot(lag_ref[t], Wlag_ref[...],
                          preferred_element_type=jnp.float32) + tfp_ref[t]
        m = jnp.mean(xt, axis=-1, keepdims=True)
        v = jnp.mean((xt - m) ** 2, axis=-1, keepdims=True)
        xt = (xt - m) * lax.rsqrt(v + 1e-5) * lng_ref[...] + lnb_ref[...]
        xt = _elu(xt)
        h = jnp.dot(xt, g1W_ref[...], preferred_element_type=jnp.float32)
        e8 = jnp.dot(h, A1_ref[...], preferred_element_type=jnp.float32)
        hs.append(h)
        es_c.append(e8[:, :4])
        ed_c.append(e8[:, 4:])
    es16 = jnp.concatenate(es_c, 1)
    es_rot = jnp.concatenate([_roll_cols(es16, 2 * r) for r in range(8)], 1)
    t1_ref[...] = jnp.concatenate(hs + [es_rot], 1)
    ed1_ref[...] = jnp.concatenate(ed_c, 1)


def _pre(x, lagged, tf_part, p, A1):
    n = x.shape[0]
    nb = n // BLK
    return pl.pallas_call(
        _pre_body,
        grid=(nb,),
        in_specs=[
            pl.BlockSpec((BLK, 128), lambda i: (i, 0)),
            pl.BlockSpec((T, BLK, 3), lambda i: (0, i, 0)),
            pl.BlockSpec((T, H), lambda i: (0, 0)),
            pl.BlockSpec((H,), lambda i: (0,)),
            pl.BlockSpec((H,), lambda i: (0,)),
            pl.BlockSpec((128, H), lambda i: (0, 0)),
            pl.BlockSpec((3, H), lambda i: (0, 0)),
            pl.BlockSpec((H, 4 * H), lambda i: (0, 0)),
            pl.BlockSpec((4 * H, 8), lambda i: (0, 0)),
        ],
        out_specs=[
            pl.BlockSpec((BLK, D1 + 128), lambda i: (i, 0)),
            pl.BlockSpec((BLK, 16), lambda i: (i, 0)),
        ],
        out_shape=[
            jax.ShapeDtypeStruct((n, D1 + 128), jnp.float32),
            jax.ShapeDtypeStruct((n, 16), jnp.float32),
        ],
    )(x, lagged, tf_part, p['emb_ln_g'], p['emb_ln_b'],
      p['emb_W'][:128], p['emb_W'][144:], p['g1_W'], A1)


def _mid_body(agg_ref, s_ref, b1_ref, g2W_ref, A2_ref, t2_ref, ed2_ref):
    blk = agg_ref.shape[0]
    sall = s_ref[...]
    h2s, es_c, ed_c = [], [], []
    for t in range(T):
        a = agg_ref[:, t * 256:(t + 1) * 256]
        s4 = jnp.concatenate(
            [sall[(t * 4 + h) // 2, :, (t * 4 + h) % 2:(t * 4 + h) % 2 + 1]
             for h in range(4)], 1)
        sb = jnp.broadcast_to(s4[:, :, None], (blk, 4, 64)).reshape(blk, 256)
        xt1 = jnp.maximum(a / (sb + 1e-16) + b1_ref[...], 0.0)
        h2 = jnp.dot(xt1, g2W_ref[...], preferred_element_type=jnp.float32)
        e2 = jnp.dot(h2, A2_ref[...], preferred_element_type=jnp.float32)
        h2s.append(h2)
        es_c.append(e2[:, :1])
        ed_c.append(e2[:, 1:])
    z12 = jnp.zeros((blk, 12), jnp.float32)
    es16 = jnp.concatenate(es_c + [z12], 1)
    es_rot = jnp.concatenate(
        [_roll_cols(es16, 2 * (r % 2)) for r in range(8)], 1)
    t2_ref[...] = jnp.concatenate(h2s + [es_rot], 1)
    ed2_ref[...] = jnp.concatenate(ed_c + [z12], 1)


def _mid(agg1, s1, p, A2, n):
    nb = n // BLK
    return pl.pallas_call(
        _mid_body,
        grid=(nb,),
        in_specs=[
            pl.BlockSpec((BLK, D1), lambda i: (i, 0)),
            pl.BlockSpec((8, BLK, 16), lambda i: (0, i, 0)),
            pl.BlockSpec((4 * H,), lambda i: (0,)),
            pl.BlockSpec((4 * H, H), lambda i: (0, 0)),
            pl.BlockSpec((H, 2), lambda i: (0, 0)),
        ],
        out_specs=[
            pl.BlockSpec((BLK, D2 + 128), lambda i: (i, 0)),
            pl.BlockSpec((BLK, 16), lambda i: (i, 0)),
        ],
        out_shape=[
            jax.ShapeDtypeStruct((n, D2 + 128), jnp.float32),
            jax.ShapeDtypeStruct((n, 16), jnp.float32),
        ],
    )(agg1, s1, p['g1_b'], p['g2_W'], A2)


def _tail_body(agg_ref, s_ref, g2b_ref, c_ref, fuW_ref, fg_ref, fb_ref,
               o1W_ref, o1b_ref, o1g_ref, o1bb_ref, o2W_ref, o2b_ref, out_ref):
    sall = s_ref[...]
    for t in range(T):
        a = agg_ref[:, t * 64:(t + 1) * 64]
        s = sall[t // 2, :, t % 2:t % 2 + 1]
        xt2 = jnp.maximum(a / (s + 1e-16) + g2b_ref[...], 0.0)
        y = jnp.dot(xt2, fuW_ref[...], preferred_element_type=jnp.float32) + c_ref[t]
        m = jnp.mean(y, axis=-1, keepdims=True)
        v = jnp.mean((y - m) ** 2, axis=-1, keepdims=True)
        y = (y - m) * lax.rsqrt(v + 1e-5) * fg_ref[...] + fb_ref[...]
        y = _elu(y)
        z = jnp.dot(y, o1W_ref[...], preferred_element_type=jnp.float32) + o1b_ref[...]
        m = jnp.mean(z, axis=-1, keepdims=True)
        v = jnp.mean((z - m) ** 2, axis=-1, keepdims=True)
        z = (z - m) * lax.rsqrt(v + 1e-5) * o1g_ref[...] + o1bb_ref[...]
        z = _elu(z)
        o = jnp.dot(z, o2W_ref[...], preferred_element_type=jnp.float32)
        out_ref[0, t] = o[:, 0] + o2b_ref[0]


def _tail(agg2, s2, consts, p, n):
    nb = n // BLK
    out = pl.pallas_call(
        _tail_body,
        grid=(nb,),
        in_specs=[
            pl.BlockSpec((BLK, D2), lambda i: (i, 0)),
            pl.BlockSpec((2, BLK, 16), lambda i: (0, i, 0)),
            pl.BlockSpec((H,), lambda i: (0,)),
            pl.BlockSpec((T, H), lambda i: (0, 0)),
            pl.BlockSpec((H, H), lambda i: (0, 0)),
            pl.BlockSpec((H,), lambda i: (0,)),
            pl.BlockSpec((H,), lambda i: (0,)),
            pl.BlockSpec((H, H // 2), lambda i: (0, 0)),
            pl.BlockSpec((H // 2,), lambda i: (0,)),
            pl.BlockSpec((H // 2,), lambda i: (0,)),
            pl.BlockSpec((H // 2,), lambda i: (0,)),
            pl.BlockSpec((H // 2, 1), lambda i: (0, 0)),
            pl.BlockSpec((1,), lambda i: (0,)),
        ],
        out_specs=pl.BlockSpec((1, T, BLK), lambda i: (i, 0, 0)),
        out_shape=jax.ShapeDtypeStruct((nb, T, BLK), jnp.float32),
    )(agg2, s2, p['g2_b'], consts, p['fu_W'][:H], p['fu_ln_g'], p['fu_ln_b'],
      p['o1_W'], p['o1_b'], p['o1_ln_g'], p['o1_ln_b'], p['o2_W'], p['o2_b'])
    return out.transpose(1, 0, 2).reshape(T, n)


def _make_gat_agg(NCB):
    RPN = NCB + 1
    NG = 2 * (16 // NCB)
    mesh = plsc.VectorSubcoreMesh(core_axis_name="c", subcore_axis_name="s")

    @functools.partial(
        pl.kernel,
        out_type=(jax.ShapeDtypeStruct((NPAD, NCB, 128), jnp.float32),
                  jax.ShapeDtypeStruct((NCB, NPAD, 16), jnp.float32)),
        mesh=mesh,
        scratch_types=[
            pltpu.VMEM((KE,), jnp.int32),
            pltpu.VMEM((KE,), jnp.int32),
            pltpu.VMEM((KE,), jnp.int32),
            pltpu.VMEM((KE,), jnp.int32),
            pltpu.VMEM((KE, 128), jnp.float32),
            pltpu.VMEM((KE, 128), jnp.float32),
            pltpu.VMEM((C, 16), jnp.float32),
            pltpu.VMEM((224,), jnp.int32),
            pltpu.VMEM((CD, 128), jnp.float32),
            pltpu.VMEM((CD, 16), jnp.float32),
            pltpu.SemaphoreType.DMA,
            pltpu.SemaphoreType.DMA,
        ],
    )
    def k(table, ed_t, srcs, dsts, offs, out, out_s,
          src_v, dst_v, idx_v, idxe_v, rows_v, es_v, ed_v, offs_v,
          acc, s_acc, sem, sem2):
        cid = lax.axis_index("c")
        sid = lax.axis_index("s")
        cb = sid % NCB
        eh = sid // NCB
        gidx = cid * (16 // NCB) + eh
        lanes = jnp.arange(16, dtype=jnp.int32)
        z16 = jnp.zeros((16,), jnp.float32)
        d01 = jnp.where(lanes < 2, 1.0, 0.0)
        pltpu.sync_copy(offs.at[pl.ds(0, 224)], offs_v)
        nch_mine = (NCH + NG - 1 - gidx) // NG

        def chunk(i, carry):
            ch = i * NG + gidx
            cbase = ch * C

            def zr(r, c):
                for q in range(8):
                    acc[r, pl.ds(q * 16, 16)] = z16
                s_acc[r] = z16
                return c

            lax.fori_loop(0, C, zr, 0)
            pltpu.sync_copy(ed_t.at[sid, pl.ds(cbase, C)], ed_v)
            ovec = offs_v[pl.ds(ch, 16)]
            o0 = ovec[0]
            o1 = ovec[1]
            w0 = (o0 // 8) * 8
            nb = (o1 - w0 + KE - 1) // KE

            def batch(b, c2):
                base = pl.multiple_of(w0 + b * KE, 8)
                pltpu.sync_copy(srcs.at[pl.ds(base, KE)], src_v)
                pltpu.sync_copy(dsts.at[pl.ds(base, KE)], dst_v)
                for g in range(KE // 16):
                    sv = src_v[pl.ds(g * 16, 16)]
                    idx_v[pl.ds(g * 16, 16)] = sv * RPN + cb
                    idxe_v[pl.ds(g * 16, 16)] = sv * RPN + NCB
                cp = pltpu.make_async_copy(table.at[idx_v], rows_v, sem)
                cp2 = pltpu.make_async_copy(table.at[idxe_v], es_v, sem2)
                cp.start()
                cp2.start()
                cp.wait()
                cp2.wait()

                def group(g, c3):
                    dv = dst_v[pl.ds(g * 16, 16)] - cbase
                    pos = base + g * 16 + lanes
                    m = (dv >= 0) & (dv < C) & (pos < o1)
                    ld16 = jnp.where(m, dv, C)
                    ldc = jnp.minimum(ld16, C - 1)
                    for l in range(16):
                        kk = g * 16 + l
                        r = ld16[l]
                        rc = ldc[l]
                        ev = es_v[kk, pl.ds(0, 16)] + ed_v[rc]
                        pv = jnp.exp(jnp.maximum(ev, 0.2 * ev))
                        s0 = pv[0]
                        s1 = pv[1]
                        for q in range(4):
                            sl = pl.ds(q * 16, 16)
                            plsc.addupdate(acc.at[r, sl], rows_v[kk, sl] * s0)
                        for q in range(4, 8):
                            sl = pl.ds(q * 16, 16)
                            plsc.addupdate(acc.at[r, sl], rows_v[kk, sl] * s1)
                        plsc.addupdate(s_acc.at[r], pv * d01)
                    return c3

                lax.fori_loop(0, KE // 16, group, 0)
                return c2

            lax.fori_loop(0, nb, batch, 0)
            pltpu.sync_copy(acc.at[pl.ds(0, C)], out.at[pl.ds(cbase, C), cb])
            pltpu.sync_copy(s_acc.at[pl.ds(0, C)], out_s.at[cb, pl.ds(cbase, C)])
            return carry

        lax.fori_loop(0, nch_mine, chunk, 0)

    return k


def kernel(x, edge_index, edge_attr, temporal_features, lagged_targets, params):
    p = params
    src = edge_index[0]
    dst = edge_index[1]
    n = x.shape[0]

    tf = temporal_features
    tp0 = jax.nn.elu(_ln(tf @ p['tn_W'] + p['tn_b'], p['tn_ln_g'], p['tn_ln_b']))
    bank = None
    tps = []
    for t in range(T):
        cur, bank = _temporal_mem(tp0[t:t + 1], bank, p)
        tps.append(cur[0])
    tp_all = jnp.stack(tps)
    consts = tp_all @ p['fu_W'][H:] + p['fu_b']

    tf_part = tf @ p['emb_W'][128:144] + p['emb_b']
    idx = jnp.arange(256)
    hd_idx = idx // 64
    ch_idx = idx % 64
    A1 = jnp.zeros((256, 8), jnp.float32)
    A1 = A1.at[idx, hd_idx].set(p['g1_as'][hd_idx, ch_idx])
    A1 = A1.at[idx, hd_idx + 4].set(p['g1_ad'][hd_idx, ch_idx])
    A2 = jnp.concatenate([p['g2_as'].T, p['g2_ad'].T], axis=1)

    dst_s, perm = lax.sort_key_val(dst, jnp.arange(dst.shape[0], dtype=jnp.int32))
    src_s = src[perm]
    pad = jnp.zeros((128,), jnp.int32)
    srcs_p = jnp.concatenate([src_s, pad])
    dsts_p = jnp.concatenate([dst_s, pad])
    bounds = jnp.arange(NCH + 1, dtype=jnp.int32) * C
    offs = jnp.searchsorted(dst_s, bounds).astype(jnp.int32)
    offs = jnp.concatenate([offs, jnp.full((224 - NCH - 1,), dst.shape[0], jnp.int32)])

    table1, ed1 = _pre(x, lagged_targets, tf_part, p, A1)
    ed1p = jnp.zeros((NPAD, 16), jnp.float32).at[:n].set(ed1)
    ed1r = jnp.stack([jnp.roll(ed1p, -2 * (s % 8), axis=1) for s in range(16)],
                     axis=0)
    agg1, s1 = _make_gat_agg(8)(
        table1.reshape(n * 9, 128), ed1r, srcs_p, dsts_p, offs)
    table2, ed2 = _mid(agg1.reshape(NPAD, D1)[:n], s1[:, :n], p, A2, n)
    ed2p = jnp.zeros((NPAD, 16), jnp.float32).at[:n].set(ed2)
    ed2r = jnp.stack([jnp.roll(ed2p, -2 * (s % 2), axis=1) for s in range(16)],
                     axis=0)
    agg2, s2 = _make_gat_agg(2)(
        table2.reshape(n * 3, 128), ed2r, srcs_p, dsts_p, offs)
    return _tail(agg2.reshape(NPAD, D2)[:n], s2[:, :n], consts, p, n)


# speedup vs baseline: 21.0800x; 1.0631x over previous
import functools

import jax
import jax.numpy as jnp
from jax import lax
from jax.experimental import pallas as pl
from jax.experimental.pallas import tpu as pltpu
from jax.experimental.pallas import tpu_sc as plsc

N_LAG_E = 3
H = 64
T = 4
BLK = 1000

C = 256
CD = C + 8
NCH = 196
NPAD = NCH * C

D1 = 1024
D2 = 256
KE = 32


def _ln(x, g, b):
    m = jnp.mean(x, axis=-1, keepdims=True)
    v = jnp.var(x, axis=-1, keepdims=True)
    return (x - m) / jnp.sqrt(v + 1e-5) * g + b


def _gru(seq, Wi, Wh, bi, bh):
    n, L, hh = seq.shape
    h = jnp.zeros((n, hh), dtype=seq.dtype)
    outs = []
    for t in range(L):
        gi = seq[:, t, :] @ Wi + bi
        gh = h @ Wh + bh
        ir, iz, inn = jnp.split(gi, 3, axis=-1)
        hr, hz, hn = jnp.split(gh, 3, axis=-1)
        r = jax.nn.sigmoid(ir + hr)
        z = jax.nn.sigmoid(iz + hz)
        ng = jnp.tanh(inn + r * hn)
        h = (1.0 - z) * ng + z * h
        outs.append(h)
    return jnp.stack(outs, axis=1)


def _temporal_mem(cur, bank, p):
    if bank is None:
        return cur, jnp.zeros((cur.shape[0], N_LAG_E, cur.shape[1]), cur.dtype)
    out0 = _gru(bank, p['gru_Wi0'], p['gru_Wh0'], p['gru_bi0'], p['gru_bh0'])
    out1 = _gru(out0, p['gru_Wi1'], p['gru_Wh1'], p['gru_bi1'], p['gru_bh1'])
    exp_cur = jnp.broadcast_to(cur[:, None, :], out1.shape)
    ai = jnp.concatenate([exp_cur, out1], axis=-1)
    hid = jnp.tanh(ai @ p['ma_W1'] + p['ma_b1'])
    logits = hid @ p['ma_W2'] + p['ma_b2']
    w = jax.nn.softmax(logits, axis=1)
    weighted = jnp.sum(out1 * w, axis=1)
    new_bank = jnp.concatenate([bank[:, 1:, :], cur[:, None, :]], axis=1)
    return cur + weighted, new_bank


def _elu(x):
    return jnp.where(x > 0, x, jnp.exp(x) - 1.0)


def _roll_cols(x, k):
    k = k % 16
    if k == 0:
        return x
    return jnp.concatenate([x[:, k:], x[:, :k]], axis=1)


def _pre_body(x_ref, lag_ref, tfp_ref, lng_ref, lnb_ref, embWx_ref, Wlag_ref,
              g1W_ref, A1_ref, t1_ref, ed1_ref):
    xw = jnp.dot(x_ref[...], embWx_ref[...], preferred_element_type=jnp.float32)
    hs, es_c, ed_c = [], [], []
    for t in range(T):
        xt = xw + jnp.dot(lag_ref[t], Wlag_ref[...],
                          preferred_element_type=jnp.float32) + tfp_ref[t]
        m = jnp.mean(xt, axis=-1, keepdims=True)
        v = jnp.mean((xt - m) ** 2, axis=-1, keepdims=True)
        xt = (xt - m) * lax.rsqrt(v + 1e-5) * lng_ref[...] + lnb_ref[...]
        xt = _elu(xt)
        h = jnp.dot(xt, g1W_ref[...], preferred_element_type=jnp.float32)
        e8 = jnp.dot(h, A1_ref[...], preferred_element_type=jnp.float32)
        hs.append(h)
        es_c.append(e8[:, :4])
        ed_c.append(e8[:, 4:])
    es16 = jnp.concatenate(es_c, 1)
    es_rot = jnp.concatenate([_roll_cols(es16, 2 * r) for r in range(8)], 1)
    t1_ref[...] = jnp.concatenate(hs + [es_rot], 1)
    ed1_ref[...] = jnp.concatenate(ed_c, 1)


def _pre(x, lagged, tf_part, p, A1):
    n = x.shape[0]
    nb = n // BLK
    return pl.pallas_call(
        _pre_body,
        grid=(nb,),
        in_specs=[
            pl.BlockSpec((BLK, 128), lambda i: (i, 0)),
            pl.BlockSpec((T, BLK, 3), lambda i: (0, i, 0)),
            pl.BlockSpec((T, H), lambda i: (0, 0)),
            pl.BlockSpec((H,), lambda i: (0,)),
            pl.BlockSpec((H,), lambda i: (0,)),
            pl.BlockSpec((128, H), lambda i: (0, 0)),
            pl.BlockSpec((3, H), lambda i: (0, 0)),
            pl.BlockSpec((H, 4 * H), lambda i: (0, 0)),
            pl.BlockSpec((4 * H, 8), lambda i: (0, 0)),
        ],
        out_specs=[
            pl.BlockSpec((BLK, D1 + 128), lambda i: (i, 0)),
            pl.BlockSpec((BLK, 16), lambda i: (i, 0)),
        ],
        out_shape=[
            jax.ShapeDtypeStruct((n, D1 + 128), jnp.float32),
            jax.ShapeDtypeStruct((n, 16), jnp.float32),
        ],
    )(x, lagged, tf_part, p['emb_ln_g'], p['emb_ln_b'],
      p['emb_W'][:128], p['emb_W'][144:], p['g1_W'], A1)


def _mid_body(agg_ref, s_ref, b1_ref, g2W_ref, A2_ref, t2_ref, ed2_ref):
    blk = agg_ref.shape[0]
    sall = s_ref[...]
    h2s, es_c, ed_c = [], [], []
    for t in range(T):
        a = agg_ref[:, t * 256:(t + 1) * 256]
        s4 = jnp.concatenate(
            [sall[(t * 4 + h) // 2, :, (t * 4 + h) % 2:(t * 4 + h) % 2 + 1]
             for h in range(4)], 1)
        sb = jnp.broadcast_to(s4[:, :, None], (blk, 4, 64)).reshape(blk, 256)
        xt1 = jnp.maximum(a / (sb + 1e-16) + b1_ref[...], 0.0)
        h2 = jnp.dot(xt1, g2W_ref[...], preferred_element_type=jnp.float32)
        e2 = jnp.dot(h2, A2_ref[...], preferred_element_type=jnp.float32)
        h2s.append(h2)
        es_c.append(e2[:, :1])
        ed_c.append(e2[:, 1:])
    z12 = jnp.zeros((blk, 12), jnp.float32)
    es16 = jnp.concatenate(es_c + [z12], 1)
    es_rot = jnp.concatenate(
        [_roll_cols(es16, 2 * (r % 2)) for r in range(8)], 1)
    t2_ref[...] = jnp.concatenate(h2s + [es_rot], 1)
    ed2_ref[...] = jnp.concatenate(ed_c + [z12], 1)


def _mid(agg1, s1, p, A2, n):
    nb = n // BLK
    return pl.pallas_call(
        _mid_body,
        grid=(nb,),
        in_specs=[
            pl.BlockSpec((BLK, D1), lambda i: (i, 0)),
            pl.BlockSpec((8, BLK, 16), lambda i: (0, i, 0)),
            pl.BlockSpec((4 * H,), lambda i: (0,)),
            pl.BlockSpec((4 * H, H), lambda i: (0, 0)),
            pl.BlockSpec((H, 2), lambda i: (0, 0)),
        ],
        out_specs=[
            pl.BlockSpec((BLK, D2 + 128), lambda i: (i, 0)),
            pl.BlockSpec((BLK, 16), lambda i: (i, 0)),
        ],
        out_shape=[
            jax.ShapeDtypeStruct((n, D2 + 128), jnp.float32),
            jax.ShapeDtypeStruct((n, 16), jnp.float32),
        ],
    )(agg1, s1, p['g1_b'], p['g2_W'], A2)


def _tail_body(agg_ref, s_ref, g2b_ref, c_ref, fuW_ref, fg_ref, fb_ref,
               o1W_ref, o1b_ref, o1g_ref, o1bb_ref, o2W_ref, o2b_ref, out_ref):
    sall = s_ref[...]
    for t in range(T):
        a = agg_ref[:, t * 64:(t + 1) * 64]
        s = sall[t // 2, :, t % 2:t % 2 + 1]
        xt2 = jnp.maximum(a / (s + 1e-16) + g2b_ref[...], 0.0)
        y = jnp.dot(xt2, fuW_ref[...], preferred_element_type=jnp.float32) + c_ref[t]
        m = jnp.mean(y, axis=-1, keepdims=True)
        v = jnp.mean((y - m) ** 2, axis=-1, keepdims=True)
        y = (y - m) * lax.rsqrt(v + 1e-5) * fg_ref[...] + fb_ref[...]
        y = _elu(y)
        z = jnp.dot(y, o1W_ref[...], preferred_element_type=jnp.float32) + o1b_ref[...]
        m = jnp.mean(z, axis=-1, keepdims=True)
        v = jnp.mean((z - m) ** 2, axis=-1, keepdims=True)
        z = (z - m) * lax.rsqrt(v + 1e-5) * o1g_ref[...] + o1bb_ref[...]
        z = _elu(z)
        o = jnp.dot(z, o2W_ref[...], preferred_element_type=jnp.float32)
        out_ref[0, t] = o[:, 0] + o2b_ref[0]


def _tail(agg2, s2, consts, p, n):
    nb = n // BLK
    out = pl.pallas_call(
        _tail_body,
        grid=(nb,),
        in_specs=[
            pl.BlockSpec((BLK, D2), lambda i: (i, 0)),
            pl.BlockSpec((2, BLK, 16), lambda i: (0, i, 0)),
            pl.BlockSpec((H,), lambda i: (0,)),
            pl.BlockSpec((T, H), lambda i: (0, 0)),
            pl.BlockSpec((H, H), lambda i: (0, 0)),
            pl.BlockSpec((H,), lambda i: (0,)),
            pl.BlockSpec((H,), lambda i: (0,)),
            pl.BlockSpec((H, H // 2), lambda i: (0, 0)),
            pl.BlockSpec((H // 2,), lambda i: (0,)),
            pl.BlockSpec((H // 2,), lambda i: (0,)),
            pl.BlockSpec((H // 2,), lambda i: (0,)),
            pl.BlockSpec((H // 2, 1), lambda i: (0, 0)),
            pl.BlockSpec((1,), lambda i: (0,)),
        ],
        out_specs=pl.BlockSpec((1, T, BLK), lambda i: (i, 0, 0)),
        out_shape=jax.ShapeDtypeStruct((nb, T, BLK), jnp.float32),
    )(agg2, s2, p['g2_b'], consts, p['fu_W'][:H], p['fu_ln_g'], p['fu_ln_b'],
      p['o1_W'], p['o1_b'], p['o1_ln_g'], p['o1_ln_b'], p['o2_W'], p['o2_b'])
    return out.transpose(1, 0, 2).reshape(T, n)


def _make_gat_agg(NCB):
    RPN = NCB + 1
    NG = 2 * (16 // NCB)
    mesh = plsc.VectorSubcoreMesh(core_axis_name="c", subcore_axis_name="s")

    @functools.partial(
        pl.kernel,
        out_type=(jax.ShapeDtypeStruct((NPAD, NCB, 128), jnp.float32),
                  jax.ShapeDtypeStruct((NCB, NPAD, 16), jnp.float32)),
        mesh=mesh,
        scratch_types=[
            pltpu.VMEM((KE,), jnp.int32),
            pltpu.VMEM((2, KE), jnp.int32),
            pltpu.VMEM((2, KE), jnp.int32),
            pltpu.VMEM((2, KE), jnp.int32),
            pltpu.VMEM((2, KE, 128), jnp.float32),
            pltpu.VMEM((2, KE, 128), jnp.float32),
            pltpu.VMEM((C, 16), jnp.float32),
            pltpu.VMEM((224,), jnp.int32),
            pltpu.VMEM((CD, 128), jnp.float32),
            pltpu.VMEM((CD, 16), jnp.float32),
            pltpu.SemaphoreType.DMA,
            pltpu.SemaphoreType.DMA,
            pltpu.SemaphoreType.DMA,
            pltpu.SemaphoreType.DMA,
        ],
    )
    def k(table, ed_t, srcs, dsts, offs, out, out_s,
          src_v, dst_v, idx_v, idxe_v, rows_v, es_v, ed_v, offs_v,
          acc, s_acc, semA0, semB0, semA1, semB1):
        sems = ((semA0, semB0), (semA1, semB1))
        cid = lax.axis_index("c")
        sid = lax.axis_index("s")
        cb = sid % NCB
        eh = sid // NCB
        gidx = cid * (16 // NCB) + eh
        lanes = jnp.arange(16, dtype=jnp.int32)
        z16 = jnp.zeros((16,), jnp.float32)
        d01 = jnp.where(lanes < 2, 1.0, 0.0)
        pltpu.sync_copy(offs.at[pl.ds(0, 224)], offs_v)
        nch_mine = (NCH + NG - 1 - gidx) // NG

        def chunk(i, carry):
            ch = i * NG + gidx
            cbase = ch * C

            def zr(r, c):
                for q in range(8):
                    acc[r, pl.ds(q * 16, 16)] = z16
                s_acc[r] = z16
                return c

            lax.fori_loop(0, C, zr, 0)
            pltpu.sync_copy(ed_t.at[sid, pl.ds(cbase, C)], ed_v)
            ovec = offs_v[pl.ds(ch, 16)]
            o0 = ovec[0]
            o1 = ovec[1]
            w0 = (o0 // 8) * 8
            nb = (o1 - w0 + KE - 1) // KE

            def fire(b, sl):
                base = pl.multiple_of(w0 + b * KE, 8)
                pltpu.sync_copy(srcs.at[pl.ds(base, KE)], src_v)
                pltpu.sync_copy(dsts.at[pl.ds(base, KE)], dst_v.at[sl])
                for g in range(KE // 16):
                    sv = src_v[pl.ds(g * 16, 16)]
                    idx_v[sl, pl.ds(g * 16, 16)] = sv * RPN + cb
                    idxe_v[sl, pl.ds(g * 16, 16)] = sv * RPN + NCB
                pltpu.make_async_copy(table.at[idx_v.at[sl]], rows_v.at[sl],
                                      sems[sl][0]).start()
                pltpu.make_async_copy(table.at[idxe_v.at[sl]], es_v.at[sl],
                                      sems[sl][1]).start()

            def process(b, sl):
                base = pl.multiple_of(w0 + b * KE, 8)
                pltpu.make_async_copy(table.at[idx_v.at[sl]], rows_v.at[sl],
                                      sems[sl][0]).wait()
                pltpu.make_async_copy(table.at[idxe_v.at[sl]], es_v.at[sl],
                                      sems[sl][1]).wait()

                def group(g, c3):
                    dv = dst_v[sl, pl.ds(g * 16, 16)] - cbase
                    pos = base + g * 16 + lanes
                    m = (dv >= 0) & (dv < C) & (pos < o1)
                    ld16 = jnp.where(m, dv, C)
                    ldc = jnp.minimum(ld16, C - 1)
                    for l in range(16):
                        kk = g * 16 + l
                        r = ld16[l]
                        rc = ldc[l]
                        ev = es_v[sl, kk, pl.ds(0, 16)] + ed_v[rc]
                        pv = jnp.exp(jnp.maximum(ev, 0.2 * ev))
                        s0 = pv[0]
                        s1 = pv[1]
                        for q in range(4):
                            s = pl.ds(q * 16, 16)
                            plsc.addupdate(acc.at[r, s], rows_v[sl, kk, s] * s0)
                        for q in range(4, 8):
                            s = pl.ds(q * 16, 16)
                            plsc.addupdate(acc.at[r, s], rows_v[sl, kk, s] * s1)
                        plsc.addupdate(s_acc.at[r], pv * d01)
                    return c3

                lax.fori_loop(0, KE // 16, group, 0)

            @pl.when(nb > 0)
            def _():
                fire(0, 0)

            def pair(j, c2):
                b0 = 2 * j
                b1 = 2 * j + 1

                @pl.when(b1 < nb)
                def _():
                    fire(b1, 1)

                process(b0, 0)

                @pl.when(b1 + 1 < nb)
                def _():
                    fire(b1 + 1, 0)

                @pl.when(b1 < nb)
                def _():
                    process(b1, 1)

                return c2

            lax.fori_loop(0, (nb + 1) // 2, pair, 0)
            pltpu.sync_copy(acc.at[pl.ds(0, C)], out.at[pl.ds(cbase, C), cb])
            pltpu.sync_copy(s_acc.at[pl.ds(0, C)], out_s.at[cb, pl.ds(cbase, C)])
            return carry

        lax.fori_loop(0, nch_mine, chunk, 0)

    return k


def kernel(x, edge_index, edge_attr, temporal_features, lagged_targets, params):
    p = params
    src = edge_index[0]
    dst = edge_index[1]
    n = x.shape[0]

    tf = temporal_features
    tp0 = jax.nn.elu(_ln(tf @ p['tn_W'] + p['tn_b'], p['tn_ln_g'], p['tn_ln_b']))
    bank = None
    tps = []
    for t in range(T):
        cur, bank = _temporal_mem(tp0[t:t + 1], bank, p)
        tps.append(cur[0])
    tp_all = jnp.stack(tps)
    consts = tp_all @ p['fu_W'][H:] + p['fu_b']

    tf_part = tf @ p['emb_W'][128:144] + p['emb_b']
    idx = jnp.arange(256)
    hd_idx = idx // 64
    ch_idx = idx % 64
    A1 = jnp.zeros((256, 8), jnp.float32)
    A1 = A1.at[idx, hd_idx].set(p['g1_as'][hd_idx, ch_idx])
    A1 = A1.at[idx, hd_idx + 4].set(p['g1_ad'][hd_idx, ch_idx])
    A2 = jnp.concatenate([p['g2_as'].T, p['g2_ad'].T], axis=1)

    dst_s, perm = lax.sort_key_val(dst, jnp.arange(dst.shape[0], dtype=jnp.int32))
    src_s = src[perm]
    pad = jnp.zeros((128,), jnp.int32)
    srcs_p = jnp.concatenate([src_s, pad])
    dsts_p = jnp.concatenate([dst_s, pad])
    bounds = jnp.arange(NCH + 1, dtype=jnp.int32) * C
    offs = jnp.searchsorted(dst_s, bounds).astype(jnp.int32)
    offs = jnp.concatenate([offs, jnp.full((224 - NCH - 1,), dst.shape[0], jnp.int32)])

    table1, ed1 = _pre(x, lagged_targets, tf_part, p, A1)
    ed1p = jnp.zeros((NPAD, 16), jnp.float32).at[:n].set(ed1)
    ed1r = jnp.stack([jnp.roll(ed1p, -2 * (s % 8), axis=1) for s in range(16)],
                     axis=0)
    agg1, s1 = _make_gat_agg(8)(
        table1.reshape(n * 9, 128), ed1r, srcs_p, dsts_p, offs)
    table2, ed2 = _mid(agg1.reshape(NPAD, D1)[:n], s1[:, :n], p, A2, n)
    ed2p = jnp.zeros((NPAD, 16), jnp.float32).at[:n].set(ed2)
    ed2r = jnp.stack([jnp.roll(ed2p, -2 * (s % 2), axis=1) for s in range(16)],
                     axis=0)
    agg2, s2 = _make_gat_agg(2)(
        table2.reshape(n * 3, 128), ed2r, srcs_p, dsts_p, offs)
    return _tail(agg2.reshape(NPAD, D2)[:n], s2[:, :n], consts, p, n)
